# baseline (speedup 1.0000x reference)
import functools

import jax
import jax.numpy as jnp
from jax import lax
from jax.experimental import pallas as pl
from jax.experimental.pallas import tpu as pltpu
from jax.experimental.pallas import tpu_sc as plsc

N_TOKENS = 5
VOCAB = 100000
EMBED_DIM = 64
BATCH = 4096
SEQ_LEN = 200

_NC, _NS = 2, 16
_NW = _NC * _NS
_BPW = BATCH // _NW
_HSEQ = SEQ_LEN // 2
_LANES = 16
_NBUF = 4
_NGRP = _BPW // _NBUF

_mesh = plsc.VectorSubcoreMesh(core_axis_name="c", subcore_axis_name="s")


@functools.partial(
    pl.kernel,
    mesh=_mesh,
    compiler_params=pltpu.CompilerParams(use_tc_tiling_on_sc=False),
    out_type=jax.ShapeDtypeStruct((BATCH * _HSEQ, 2 * EMBED_DIM), jnp.float32),
    scratch_types=[
        pltpu.VMEM((_BPW, 2, _HSEQ), jnp.int32),
        pltpu.VMEM((_BPW * N_TOKENS, EMBED_DIM), jnp.float32),
        pltpu.VMEM((N_TOKENS, EMBED_DIM), jnp.float32),
        pltpu.VMEM((N_TOKENS, EMBED_DIM), jnp.float32),
        pltpu.VMEM((_NBUF, 2, _HSEQ, EMBED_DIM), jnp.float32),
        [pltpu.SemaphoreType.DMA] * _NBUF,
        [pltpu.SemaphoreType.DMA] * _NBUF,
        pltpu.SemaphoreType.DMA,
    ],
)
def _sc_embed(tok_hbm, wte_hbm, samp_hbm, var_hbm, avg_hbm, out_hbm,
              idx_v, samp_v, var_v, avg_v, blk_v, gsems, wsems, ssem):
    wid = lax.axis_index("s") * _NC + lax.axis_index("c")
    b0 = wid * _BPW

    d0 = pltpu.async_copy(tok_hbm.at[pl.ds(b0, _BPW)], idx_v, ssem)
    d1 = pltpu.async_copy(samp_hbm.at[pl.ds(b0 * N_TOKENS, _BPW * N_TOKENS)],
                          samp_v, ssem)
    d2 = pltpu.async_copy(var_hbm, var_v, ssem)
    d3 = pltpu.async_copy(avg_hbm, avg_v, ssem)
    d0.wait()
    d1.wait()
    d2.wait()
    d3.wait()

    def start_gather(r, slot):
        pltpu.async_copy(wte_hbm.at[idx_v.at[r, 0]],
                         blk_v.at[slot, 0], gsems[slot])
        pltpu.async_copy(wte_hbm.at[idx_v.at[r, 1]],
                         blk_v.at[slot, 1], gsems[slot])

    def wait_gather(slot):
        pltpu.make_async_copy(out_hbm.at[pl.ds(0, _HSEQ)],
                              blk_v.at[slot], gsems[slot]).wait()

    def wait_write(slot):
        pltpu.make_async_copy(blk_v.at[slot],
                              out_hbm.at[pl.ds(0, _HSEQ)], wsems[slot]).wait()

    for r in range(_NBUF - 1):
        start_gather(r, r)

    def body(g, carry):
        for b in range(_NBUF):
            i = g * _NBUF + b
            prev = (b - 1) % _NBUF
            if b == 0:
                @pl.when(g >= 1)
                def _():
                    wait_write(prev)
            else:
                wait_write(prev)
            nxt = i + _NBUF - 1
            if b == 0:
                start_gather(nxt, prev)
            else:
                @pl.when(nxt < _BPW)
                def _():
                    start_gather(nxt, prev)
            wait_gather(b)
            for j in range(N_TOKENS):
                for c in range(EMBED_DIM // _LANES):
                    sl = pl.ds(c * _LANES, _LANES)
                    blk_v[b, 0, j, sl] = (samp_v[i * N_TOKENS + j, sl]
                                          * var_v[j, sl] + avg_v[j, sl])
            base = (b0 + i) * _HSEQ
            pltpu.async_copy(blk_v.at[b, 0],
                             out_hbm.at[pl.ds(base, _HSEQ),
                                        pl.ds(0, EMBED_DIM)], wsems[b])
            pltpu.async_copy(blk_v.at[b, 1],
                             out_hbm.at[pl.ds(base, _HSEQ),
                                        pl.ds(EMBED_DIM, EMBED_DIM)], wsems[b])
        return carry

    lax.fori_loop(0, _NGRP, body, 0)
    wait_write(_NBUF - 1)


def kernel(tokens, wte, avg, var):
    samp2 = jax.random.normal(jax.random.key(42),
                              (BATCH * N_TOKENS, EMBED_DIM), dtype=wte.dtype)
    tok3 = tokens.astype(jnp.int32).reshape(BATCH, 2, _HSEQ)
    out = _sc_embed(tok3, wte, samp2, var, avg)
    return (out.reshape(BATCH, _HSEQ, 2, EMBED_DIM)
            .transpose(0, 2, 1, 3)
            .reshape(BATCH, SEQ_LEN, EMBED_DIM))


# speedup vs baseline: 1.6473x; 1.6473x over previous
import functools

import jax
import jax.numpy as jnp
from jax import lax
from jax.experimental import pallas as pl
from jax.experimental.pallas import tpu as pltpu
from jax.experimental.pallas import tpu_sc as plsc

N_TOKENS = 5
VOCAB = 100000
EMBED_DIM = 64
BATCH = 4096
SEQ_LEN = 200

_NC, _NS = 2, 16
_NW = _NC * _NS
_BPW = BATCH // _NW
_G0 = 128
_G1 = SEQ_LEN - _G0
_LANES = 16
_NBUF = 4
_NGRP = _BPW // _NBUF

_mesh = plsc.VectorSubcoreMesh(core_axis_name="c", subcore_axis_name="s")


@functools.partial(
    pl.kernel,
    mesh=_mesh,
    compiler_params=pltpu.CompilerParams(use_tc_tiling_on_sc=False),
    out_type=jax.ShapeDtypeStruct((BATCH * SEQ_LEN, EMBED_DIM), jnp.float32),
    scratch_types=[
        pltpu.VMEM((_BPW, SEQ_LEN), jnp.int32),
        pltpu.VMEM((_BPW * N_TOKENS, EMBED_DIM), jnp.float32),
        pltpu.VMEM((N_TOKENS, EMBED_DIM), jnp.float32),
        pltpu.VMEM((N_TOKENS, EMBED_DIM), jnp.float32),
        pltpu.VMEM((_NBUF, SEQ_LEN, EMBED_DIM), jnp.float32),
        [pltpu.SemaphoreType.DMA] * _NBUF,
        [pltpu.SemaphoreType.DMA] * _NBUF,
        pltpu.SemaphoreType.DMA,
    ],
)
def _sc_embed(tok_hbm, wte_hbm, samp_hbm, var_hbm, avg_hbm, out_hbm,
              idx_v, samp_v, var_v, avg_v, blk_v, gsems, wsems, ssem):
    wid = lax.axis_index("s") * _NC + lax.axis_index("c")
    b0 = wid * _BPW

    d0 = pltpu.async_copy(tok_hbm.at[pl.ds(b0, _BPW)], idx_v, ssem)
    d1 = pltpu.async_copy(samp_hbm.at[pl.ds(b0 * N_TOKENS, _BPW * N_TOKENS)],
                          samp_v, ssem)
    d2 = pltpu.async_copy(var_hbm, var_v, ssem)
    d3 = pltpu.async_copy(avg_hbm, avg_v, ssem)
    d0.wait()
    d1.wait()
    d2.wait()
    d3.wait()

    def start_gather(r, slot):
        pltpu.async_copy(wte_hbm.at[idx_v.at[r, pl.ds(0, _G0)]],
                         blk_v.at[slot, pl.ds(0, _G0)], gsems[slot])
        pltpu.async_copy(wte_hbm.at[idx_v.at[r, pl.ds(_G0, _G1)]],
                         blk_v.at[slot, pl.ds(_G0, _G1)], gsems[slot])

    def wait_gather(slot):
        pltpu.make_async_copy(wte_hbm.at[pl.ds(0, SEQ_LEN)],
                              blk_v.at[slot], gsems[slot]).wait()

    def wait_write(slot):
        pltpu.make_async_copy(blk_v.at[slot],
                              out_hbm.at[pl.ds(0, SEQ_LEN)], wsems[slot]).wait()

    for r in range(_NBUF - 1):
        start_gather(r, r)

    def body(g, carry):
        for b in range(_NBUF):
            i = g * _NBUF + b
            prev = (b - 1) % _NBUF
            if b == 0:
                @pl.when(g >= 1)
                def _():
                    wait_write(prev)
            else:
                wait_write(prev)
            nxt = i + _NBUF - 1
            if b == 0:
                start_gather(nxt, prev)
            else:
                @pl.when(nxt < _BPW)
                def _():
                    start_gather(nxt, prev)
            wait_gather(b)
            for j in range(N_TOKENS):
                for c in range(EMBED_DIM // _LANES):
                    sl = pl.ds(c * _LANES, _LANES)
                    blk_v[b, j, sl] = (samp_v[i * N_TOKENS + j, sl]
                                       * var_v[j, sl] + avg_v[j, sl])
            pltpu.async_copy(blk_v.at[b],
                             out_hbm.at[pl.ds((b0 + i) * SEQ_LEN, SEQ_LEN)],
                             wsems[b])
        return carry

    lax.fori_loop(0, _NGRP, body, 0)
    wait_write(_NBUF - 1)


def kernel(tokens, wte, avg, var):
    samp2 = jax.random.normal(jax.random.key(42),
                              (BATCH * N_TOKENS, EMBED_DIM), dtype=wte.dtype)
    out = _sc_embed(tokens.astype(jnp.int32), wte, samp2, var, avg)
    return out.reshape(BATCH, SEQ_LEN, EMBED_DIM)
